# baseline (speedup 1.0000x reference)
import functools

import jax
import jax.numpy as jnp
from jax import lax
from jax.experimental import pallas as pl
from jax.experimental.pallas import tpu as pltpu
from jax.experimental.pallas import tpu_sc as plsc

_LANES = 16
_N_TC_FRAC_NUM, _N_TC_FRAC_DEN = 1, 2


@functools.lru_cache(maxsize=None)
def _build_sc(N: int, D: int, row0: int, n_rows: int):
    info = plsc.get_sparse_core_info()
    NC, NS = info.num_cores, info.num_subcores
    NW = NC * NS

    assert n_rows % NW == 0 and D % _LANES == 0
    rows_per_w = n_rows // NW
    C = 16
    assert rows_per_w % C == 0
    n_chunks = rows_per_w // C
    assert n_chunks % 4 == 0 and n_chunks >= 8

    mesh = plsc.VectorSubcoreMesh(core_axis_name="c", subcore_axis_name="s")

    @functools.partial(
        pl.kernel,
        mesh=mesh,
        out_type=jax.ShapeDtypeStruct((n_rows, D), jnp.float32),
        scratch_types=[
            pltpu.VMEM((rows_per_w,), jnp.int32),
            pltpu.VMEM((4, C, D), jnp.float32),
            pltpu.VMEM((2, C, D), jnp.float32),
            pltpu.SemaphoreType.DMA,
            pltpu.SemaphoreType.DMA,
            pltpu.SemaphoreType.DMA,
            pltpu.SemaphoreType.DMA,
            pltpu.SemaphoreType.DMA,
            pltpu.SemaphoreType.DMA,
            pltpu.SemaphoreType.DMA,
            pltpu.SemaphoreType.DMA,
            pltpu.SemaphoreType.DMA,
            pltpu.SemaphoreType.DMA,
        ],
    )
    def k(x_hbm, idx_hbm, w_hbm, out_hbm, idx_v, g_v, x_v,
          gs0, gs1, gs2, gs3, xs0, xs1, os0, os1, os2, os3):
        wid = lax.axis_index("s") * NC + lax.axis_index("c")
        base = row0 + wid * rows_per_w
        obase = wid * rows_per_w
        gsem = (gs0, gs1, gs2, gs3)
        xsem = (xs0, xs1)
        osem = (os0, os1, os2, os3)

        pltpu.sync_copy(idx_hbm.at[pl.ds(base, rows_per_w)], idx_v)

        def issue_in(c, bg, bx):
            pltpu.async_copy(
                w_hbm.at[idx_v.at[pl.ds(c * C, C)]], g_v.at[bg], gsem[bg])
            pltpu.async_copy(
                x_hbm.at[pl.ds(base + c * C, C), :], x_v.at[bx], xsem[bx])

        def wait_out(bg):
            pltpu.make_async_copy(g_v.at[bg], out_hbm.at[pl.ds(obase, C), :],
                                  osem[bg]).wait()

        def step(c, k_):
            bg, bx = k_ % 4, k_ % 2
            nbg, nbx = (k_ + 1) % 4, (k_ + 1) % 2

            @pl.when(c >= 3)
            def _():
                wait_out(nbg)

            @pl.when(c + 1 < n_chunks)
            def _():
                issue_in(c + 1, nbg, nbx)

            pltpu.make_async_copy(w_hbm.at[idx_v.at[pl.ds(0, C)]],
                                  g_v.at[bg], gsem[bg]).wait()
            pltpu.make_async_copy(x_hbm.at[pl.ds(base, C), :],
                                  x_v.at[bx], xsem[bx]).wait()

            @plsc.parallel_loop(0, C)
            def row_body(r):
                for j in range(D // _LANES):
                    sl = pl.ds(j * _LANES, _LANES)
                    plsc.addupdate(g_v.at[bg, r, sl], x_v[bx, r, sl])

            pltpu.async_copy(g_v.at[bg],
                             out_hbm.at[pl.ds(obase + c * C, C), :], osem[bg])

        issue_in(0, 0, 0)

        def group(i, carry):
            for k_ in range(4):
                step(4 * i + k_, k_)
            return carry

        lax.fori_loop(0, n_chunks // 4, group, 0)
        for c in (n_chunks - 3, n_chunks - 2, n_chunks - 1):
            wait_out(c % 4)

    return k


@functools.lru_cache(maxsize=None)
def _build_tc(N: int, D: int, V: int, n_rows: int):
    RB = 512
    assert n_rows % RB == 0
    nb = n_rows // RB

    def body(ids_ref, x_ref, w_ref, out_ref):
        ids = ids_ref[0, 0, :]
        oh = (ids[:, None] ==
              lax.broadcasted_iota(jnp.int32, (RB, V), 1)
              ).astype(jnp.bfloat16)
        gath = lax.dot_general(oh, w_ref[...], (((1,), (0,)), ((), ())),
                               preferred_element_type=jnp.float32)
        out_ref[...] = x_ref[...] + gath

    return pl.pallas_call(
        body,
        grid=(nb,),
        in_specs=[
            pl.BlockSpec((1, 1, RB), lambda i: (i, 0, 0)),
            pl.BlockSpec((RB, D), lambda i: (i, 0)),
            pl.BlockSpec((V, D), lambda i: (0, 0)),
        ],
        out_specs=pl.BlockSpec((RB, D), lambda i: (i, 0)),
        out_shape=jax.ShapeDtypeStruct((n_rows, D), jnp.float32),
        compiler_params=pltpu.CompilerParams(
            dimension_semantics=("arbitrary",)),
    )


def kernel(x, pos_ids, weight):
    B, L, D = x.shape
    V = weight.shape[0]
    N = B * L
    n_tc = (N * _N_TC_FRAC_NUM // _N_TC_FRAC_DEN) // 2048 * 2048
    n_sc = N - n_tc

    x_flat = x.reshape(N, D)
    idx_flat = pos_ids.reshape(N).astype(jnp.int32)
    w_bf = weight.astype(jnp.bfloat16)
    ids3 = idx_flat.reshape(N // 512, 1, 512)

    out_tc = _build_tc(N, D, V, n_tc)(ids3, x_flat, w_bf)
    out_sc = _build_sc(N, D, n_tc, n_sc)(x_flat, idx_flat, weight)
    out = jnp.concatenate([out_tc, out_sc], axis=0)
    return out.reshape(B, L, D)


# speedup vs baseline: 1.4668x; 1.4668x over previous
import functools

import jax
import jax.numpy as jnp
import numpy as np
from jax import lax
from jax.experimental import pallas as pl
from jax.experimental.pallas import tpu as pltpu
from jax.experimental.pallas import tpu_sc as plsc

_LANES = 16


@functools.lru_cache(maxsize=None)
def _build(N: int, D: int, V: int):
    info = plsc.get_sparse_core_info()
    NC, NS = info.num_cores, info.num_subcores
    NW = NC * NS

    assert N % NW == 0 and D % (2 * _LANES) == 0
    rows_per_w = N // NW
    C = 32
    assert rows_per_w % C == 0
    n_chunks = rows_per_w // C
    assert n_chunks % 4 == 0 and n_chunks >= 8

    mesh = plsc.VectorSubcoreMesh(core_axis_name="c", subcore_axis_name="s")

    @functools.partial(
        pl.kernel,
        mesh=mesh,
        out_type=jax.ShapeDtypeStruct((N, D), jnp.float32),
        scratch_types=[
            pltpu.VMEM((rows_per_w,), jnp.int32),
            pltpu.VMEM((2, C, D // 2), jnp.int32),
            pltpu.VMEM((4, C, D), jnp.float32),
            pltpu.SemaphoreType.DMA,
            pltpu.SemaphoreType.DMA,
            pltpu.SemaphoreType.DMA,
            pltpu.SemaphoreType.DMA,
            pltpu.SemaphoreType.DMA,
            pltpu.SemaphoreType.DMA,
            pltpu.SemaphoreType.DMA,
            pltpu.SemaphoreType.DMA,
            pltpu.SemaphoreType.DMA,
            pltpu.SemaphoreType.DMA,
        ],
    )
    def k(x_hbm, idx_hbm, w_hbm, out_hbm, idx_v, g_v, x_v,
          gs0, gs1, xs0, xs1, xs2, xs3, os0, os1, os2, os3):
        wid = lax.axis_index("s") * NC + lax.axis_index("c")
        base = wid * rows_per_w
        gsem = (gs0, gs1)
        xsem = (xs0, xs1, xs2, xs3)
        osem = (os0, os1, os2, os3)

        pltpu.sync_copy(idx_hbm.at[pl.ds(base, rows_per_w)], idx_v)

        def issue_in(c, bg, bx):
            pltpu.async_copy(
                w_hbm.at[idx_v.at[pl.ds(c * C, C)]], g_v.at[bg], gsem[bg])
            pltpu.async_copy(
                x_hbm.at[pl.ds(base + c * C, C), :], x_v.at[bx], xsem[bx])

        def wait_out(bx):
            pltpu.make_async_copy(x_v.at[bx], out_hbm.at[pl.ds(base, C), :],
                                  osem[bx]).wait()

        def step(c, k_):
            bg, bx = k_ % 2, k_ % 4
            nbg, nbx = (k_ + 1) % 2, (k_ + 1) % 4

            @pl.when(c >= 3)
            def _():
                wait_out(nbx)

            @pl.when(c + 1 < n_chunks)
            def _():
                issue_in(c + 1, nbg, nbx)

            pltpu.make_async_copy(w_hbm.at[idx_v.at[pl.ds(0, C)]],
                                  g_v.at[bg], gsem[bg]).wait()
            pltpu.make_async_copy(x_hbm.at[pl.ds(base, C), :],
                                  x_v.at[bx], xsem[bx]).wait()

            @plsc.parallel_loop(0, C)
            def row_body(r):
                for j in range(D // (2 * _LANES)):
                    gp = g_v[bg, r, pl.ds(j * _LANES, _LANES)]
                    a = lax.bitcast_convert_type(
                        lax.shift_left(gp, 16), jnp.float32)
                    b = lax.bitcast_convert_type(
                        lax.bitwise_and(gp, jnp.int32(-65536)), jnp.float32)
                    plsc.addupdate(
                        x_v.at[bx, r, pl.ds(j * 2 * _LANES, _LANES)], a)
                    plsc.addupdate(
                        x_v.at[bx, r, pl.ds(j * 2 * _LANES + _LANES, _LANES)],
                        b)

            pltpu.async_copy(x_v.at[bx],
                             out_hbm.at[pl.ds(base + c * C, C), :], osem[bx])

        issue_in(0, 0, 0)

        def group(i, carry):
            for k_ in range(4):
                step(4 * i + k_, k_)
            return carry

        lax.fori_loop(0, n_chunks // 4, group, 0)
        for c in (n_chunks - 3, n_chunks - 2, n_chunks - 1):
            wait_out(c % 4)

    return k


def _col_perm(D: int) -> np.ndarray:
    pat = np.stack([np.arange(16), np.arange(16) + 16], axis=1).reshape(32)
    return (np.arange(D) // 32) * 32 + pat[np.arange(D) % 32]


def kernel(x, pos_ids, weight):
    B, L, D = x.shape
    V = weight.shape[0]
    N = B * L
    x_flat = x.reshape(N, D)
    idx_flat = pos_ids.reshape(N).astype(jnp.int32)
    w_perm = weight.astype(jnp.bfloat16)[:, _col_perm(D)]
    w_i32 = jax.lax.bitcast_convert_type(
        w_perm.reshape(V, D // 2, 2), jnp.int32)
    out = _build(N, D, V)(x_flat, idx_flat, w_i32)
    return out.reshape(B, L, D)


# speedup vs baseline: 1.4699x; 1.0021x over previous
import functools

import jax
import jax.numpy as jnp
import numpy as np
from jax import lax
from jax.experimental import pallas as pl
from jax.experimental.pallas import tpu as pltpu
from jax.experimental.pallas import tpu_sc as plsc

_LANES = 16
_XRING = 8
_GRING = 4


@functools.lru_cache(maxsize=None)
def _build(N: int, D: int, V: int):
    info = plsc.get_sparse_core_info()
    NC, NS = info.num_cores, info.num_subcores
    NW = NC * NS

    assert N % NW == 0 and D % (2 * _LANES) == 0
    rows_per_w = N // NW
    C = 16
    assert rows_per_w % C == 0
    n_chunks = rows_per_w // C
    assert n_chunks % _XRING == 0 and n_chunks >= 2 * _XRING

    mesh = plsc.VectorSubcoreMesh(core_axis_name="c", subcore_axis_name="s")

    @functools.partial(
        pl.kernel,
        mesh=mesh,
        out_type=jax.ShapeDtypeStruct((N, D), jnp.float32),
        scratch_types=[
            pltpu.VMEM((rows_per_w,), jnp.int32),
            pltpu.VMEM((_GRING, C, D // 2), jnp.int32),
            pltpu.VMEM((_XRING, C, D), jnp.float32),
        ]
        + [pltpu.SemaphoreType.DMA] * (2 * _XRING + _GRING),
    )
    def k(x_hbm, idx_hbm, w_hbm, out_hbm, idx_v, g_v, x_v, *sems):
        gsem = sems[:_GRING]
        xsem = sems[_GRING:_GRING + _XRING]
        osem = sems[_GRING + _XRING:]
        wid = lax.axis_index("s") * NC + lax.axis_index("c")
        base = wid * rows_per_w

        pltpu.sync_copy(idx_hbm.at[pl.ds(base, rows_per_w)], idx_v)

        def issue_in(c, bg, bx):
            pltpu.async_copy(
                w_hbm.at[idx_v.at[pl.ds(c * C, C)]], g_v.at[bg], gsem[bg])
            pltpu.async_copy(
                x_hbm.at[pl.ds(base + c * C, C), :], x_v.at[bx], xsem[bx])

        def wait_out(bx):
            pltpu.make_async_copy(x_v.at[bx], out_hbm.at[pl.ds(base, C), :],
                                  osem[bx]).wait()

        def step(c, k_):
            bg, bx = k_ % _GRING, k_ % _XRING
            ng, nx = (k_ + 2) % _GRING, (k_ + 2) % _XRING

            @pl.when(c >= _XRING - 2)
            def _():
                wait_out(nx)

            @pl.when(c + 2 < n_chunks)
            def _():
                issue_in(c + 2, ng, nx)

            pltpu.make_async_copy(w_hbm.at[idx_v.at[pl.ds(0, C)]],
                                  g_v.at[bg], gsem[bg]).wait()
            pltpu.make_async_copy(x_hbm.at[pl.ds(base, C), :],
                                  x_v.at[bx], xsem[bx]).wait()

            @plsc.parallel_loop(0, C)
            def row_body(r):
                for j in range(D // (2 * _LANES)):
                    gp = g_v[bg, r, pl.ds(j * _LANES, _LANES)]
                    a = lax.bitcast_convert_type(
                        lax.shift_left(gp, 16), jnp.float32)
                    b = lax.bitcast_convert_type(
                        lax.bitwise_and(gp, jnp.int32(-65536)), jnp.float32)
                    plsc.addupdate(
                        x_v.at[bx, r, pl.ds(j * 2 * _LANES, _LANES)], a)
                    plsc.addupdate(
                        x_v.at[bx, r, pl.ds(j * 2 * _LANES + _LANES, _LANES)],
                        b)

            pltpu.async_copy(x_v.at[bx],
                             out_hbm.at[pl.ds(base + c * C, C), :], osem[bx])

        issue_in(0, 0, 0)
        issue_in(1, 1, 1)

        def group(i, carry):
            for k_ in range(_XRING):
                step(_XRING * i + k_, k_)
            return carry

        lax.fori_loop(0, n_chunks // _XRING, group, 0)
        for c in range(n_chunks - (_XRING - 2), n_chunks):
            wait_out(c % _XRING)

    return k


def _col_perm(D: int) -> np.ndarray:
    pat = np.stack([np.arange(16), np.arange(16) + 16], axis=1).reshape(32)
    return (np.arange(D) // 32) * 32 + pat[np.arange(D) % 32]


def kernel(x, pos_ids, weight):
    B, L, D = x.shape
    V = weight.shape[0]
    N = B * L
    x_flat = x.reshape(N, D)
    idx_flat = pos_ids.reshape(N).astype(jnp.int32)
    w_perm = weight.astype(jnp.bfloat16)[:, _col_perm(D)]
    w_i32 = jax.lax.bitcast_convert_type(
        w_perm.reshape(V, D // 2, 2), jnp.int32)
    out = _build(N, D, V)(x_flat, idx_flat, w_i32)
    return out.reshape(B, L, D)
